# Initial kernel scaffold; baseline (speedup 1.0000x reference)
import jax
import jax.numpy as jnp
from jax.experimental import pallas as pl


def kernel(x, router_w, expert_w, expert_b):
    raise NotImplementedError("write your pallas kernel here")


# speedup vs baseline: 4.8459x; 4.8459x over previous
import functools

import jax
import jax.numpy as jnp
from jax.experimental import pallas as pl
from jax.experimental.pallas import tpu as pltpu


def _moe_kernel(x_ref, rw_ref, ew_ref, eb_ref, out_ref, base_ref, *,
                capacity, block, experts):
    i = pl.program_id(0)

    @pl.when(i == 0)
    def _():
        base_ref[...] = jnp.zeros_like(base_ref)

    x = x_ref[...]
    logits = jnp.dot(x, rw_ref[...], preferred_element_type=jnp.float32)
    m = jnp.max(logits, axis=1, keepdims=True)
    ex = jnp.exp(logits - m)
    probs = ex / jnp.sum(ex, axis=1, keepdims=True)

    pm = jnp.max(probs, axis=1, keepdims=True)
    col = jax.lax.broadcasted_iota(jnp.int32, (block, experts), 1)
    eq = probs == pm
    first = jnp.min(jnp.where(eq, col, experts), axis=1, keepdims=True)
    onehot = (col == first).astype(jnp.float32)

    r2 = jax.lax.broadcasted_iota(jnp.int32, (block, block), 0)
    c2 = jax.lax.broadcasted_iota(jnp.int32, (block, block), 1)
    tri = (c2 <= r2).astype(jnp.float32)
    csum = jnp.dot(tri, onehot, preferred_element_type=jnp.float32)
    pos = base_ref[...] + csum - 1.0
    keep = jnp.sum(onehot * (pos < capacity).astype(jnp.float32), axis=1)
    scale = pm[:, 0] * keep
    base_ref[...] = base_ref[...] + jnp.sum(onehot, axis=0, keepdims=True)

    y = jax.lax.dot_general(x, ew_ref[...], (((1,), (1,)), ((), ())),
                            preferred_element_type=jnp.float32)
    y = y + eb_ref[...]
    out_ref[...] = y * scale[:, None]


def kernel(x, router_w, expert_w, expert_b):
    B, S, H = x.shape
    E = router_w.shape[1]
    T = B * S
    capacity = T // E * 2
    block = 256
    grid = T // block

    hidden = x.reshape(T, H)
    eb = expert_b.reshape(1, H)

    out = pl.pallas_call(
        functools.partial(_moe_kernel, capacity=capacity, block=block,
                          experts=E),
        grid=(grid,),
        in_specs=[
            pl.BlockSpec((block, H), lambda i: (i, 0)),
            pl.BlockSpec((H, E), lambda i: (0, 0)),
            pl.BlockSpec((H, H), lambda i: (0, 0)),
            pl.BlockSpec((1, H), lambda i: (0, 0)),
        ],
        out_specs=pl.BlockSpec((block, H), lambda i: (i, 0)),
        out_shape=jax.ShapeDtypeStruct((T, H), jnp.float32),
        scratch_shapes=[pltpu.VMEM((1, E), jnp.float32)],
    )(hidden, router_w, expert_w, eb)
    return out.reshape(B, S, H)
